# Initial kernel scaffold; baseline (speedup 1.0000x reference)
import functools

import jax
import jax.numpy as jnp
from jax import lax
from jax.experimental import pallas as pl
from jax.experimental.pallas import tpu as pltpu
from jax.experimental.pallas import tpu_sc as plsc

N = 10000
D = 128
E = 320000
NC = 2
NS = 16
NW = NC * NS
EPW = E // NW
CH = 80
NCHUNK = EPW // CH
RPT = N // NS
DW = 16

_mesh = plsc.VectorSubcoreMesh(core_axis_name="c", subcore_axis_name="s")


@functools.partial(
    pl.kernel,
    mesh=_mesh,
    out_type=[
        jax.ShapeDtypeStruct((NC, N, D), jnp.float32),
        jax.ShapeDtypeStruct((NC, N, DW), jnp.float32),
    ],
    scratch_types=[
        pltpu.VMEM((CH,), jnp.int32),
        pltpu.VMEM((CH,), jnp.int32),
        pltpu.VMEM((CH, D), jnp.float32),
        pltpu.VMEM((CH, DW), jnp.float32),
        pltpu.VMEM_SHARED((N, D), jnp.float32),
        pltpu.VMEM_SHARED((N, DW), jnp.float32),
        pltpu.SemaphoreType.DMA,
    ],
)
def _sc_aggregate(h_hbm, src_hbm, dst_hbm, zrow_hbm, zdeg_hbm, ones_hbm,
                  out_hbm, deg_hbm,
                  src_v, dst_v, rows_v, ones_v, acc_sh, deg_sh, sem):
    c = lax.axis_index("c")
    s = lax.axis_index("s")
    w = c * NS + s
    r0 = s * RPT

    pltpu.sync_copy(zrow_hbm, acc_sh.at[pl.ds(r0, RPT)])
    pltpu.sync_copy(zdeg_hbm, deg_sh.at[pl.ds(r0, RPT)])
    pltpu.sync_copy(ones_hbm, ones_v)
    plsc.subcore_barrier()

    base = w * EPW

    def body(i, carry):
        off = base + i * CH
        pltpu.sync_copy(src_hbm.at[pl.ds(off, CH)], src_v)
        pltpu.sync_copy(dst_hbm.at[pl.ds(off, CH)], dst_v)
        pltpu.async_copy(h_hbm.at[src_v], rows_v, sem).wait()
        pltpu.sync_copy(rows_v, acc_sh.at[dst_v], add=True)
        pltpu.sync_copy(ones_v, deg_sh.at[dst_v], add=True)
        return carry

    lax.fori_loop(0, NCHUNK, body, 0)

    plsc.subcore_barrier()
    pltpu.sync_copy(acc_sh.at[pl.ds(r0, RPT)], out_hbm.at[c, pl.ds(r0, RPT)])
    pltpu.sync_copy(deg_sh.at[pl.ds(r0, RPT)], deg_hbm.at[c, pl.ds(r0, RPT)])


def _dense_body(relu, x_ref, p_ref, d_ref, ws_ref, wn_ref, b_ref, o_ref):
    deg = d_ref[0, :, 0:1] + d_ref[1, :, 0:1]
    inv = 1.0 / jnp.maximum(deg, 1.0)
    hn = (p_ref[0] + p_ref[1]) * inv
    acc = lax.dot_general(x_ref[...], ws_ref[...], (((1,), (1,)), ((), ())),
                          preferred_element_type=jnp.float32)
    acc = acc + lax.dot_general(hn, wn_ref[...], (((1,), (1,)), ((), ())),
                                preferred_element_type=jnp.float32)
    acc = acc + b_ref[0, :][None, :]
    if relu:
        acc = jnp.maximum(acc, 0.0)
    o_ref[...] = acc


def _dense(x, p, d, W_self, W_neigh, b, relu):
    bm = 1000
    grid = (N // bm,)
    return pl.pallas_call(
        functools.partial(_dense_body, relu),
        grid=grid,
        in_specs=[
            pl.BlockSpec((bm, D), lambda i: (i, 0)),
            pl.BlockSpec((NC, bm, D), lambda i: (0, i, 0)),
            pl.BlockSpec((NC, bm, DW), lambda i: (0, i, 0)),
            pl.BlockSpec((D, D), lambda i: (0, 0)),
            pl.BlockSpec((D, D), lambda i: (0, 0)),
            pl.BlockSpec((1, D), lambda i: (0, 0)),
        ],
        out_specs=pl.BlockSpec((bm, D), lambda i: (i, 0)),
        out_shape=jax.ShapeDtypeStruct((N, D), jnp.float32),
    )(x, p, d, W_self, W_neigh, b.reshape(1, D))


def kernel(x, edge_index, W_neigh1, W_self1, b1, W_neigh2, W_self2, b2):
    ei = edge_index.astype(jnp.int32)
    src, dst = ei[0], ei[1]
    zrow = jnp.zeros((RPT, D), jnp.float32)
    zdeg = jnp.zeros((RPT, DW), jnp.float32)
    ones = jnp.ones((CH, DW), jnp.float32)

    p1, d1 = _sc_aggregate(x, src, dst, zrow, zdeg, ones)
    h1 = _dense(x, p1, d1, W_self1, W_neigh1, b1, relu=True)
    p2, d2 = _sc_aggregate(h1, src, dst, zrow, zdeg, ones)
    out = _dense(h1, p2, d2, W_self2, W_neigh2, b2, relu=False)
    return out


# speedup vs baseline: 5.4012x; 5.4012x over previous
import functools

import jax
import jax.numpy as jnp
from jax import lax
from jax.experimental import pallas as pl
from jax.experimental.pallas import tpu as pltpu
from jax.experimental.pallas import tpu_sc as plsc

N = 10000
D = 128
E = 320000
NC = 2
NS = 16
NW = NC * NS
EPW = E // NW
CH = 80
NCHUNK = EPW // CH
NP = 10240
RPT = NP // NS

_mesh = plsc.VectorSubcoreMesh(core_axis_name="c", subcore_axis_name="s")


@functools.partial(
    pl.kernel,
    mesh=_mesh,
    out_type=[
        jax.ShapeDtypeStruct((NC, NP, D), jnp.float32),
        jax.ShapeDtypeStruct((NC * NP,), jnp.float32),
    ],
    scratch_types=[
        pltpu.VMEM((CH,), jnp.int32),
        pltpu.VMEM((CH,), jnp.int32),
        pltpu.VMEM((CH, D), jnp.float32),
        pltpu.VMEM((CH,), jnp.float32),
        pltpu.VMEM_SHARED((NP, D), jnp.float32),
        pltpu.VMEM_SHARED((NP,), jnp.float32),
        pltpu.SemaphoreType.DMA,
    ],
)
def _sc_aggregate(h_hbm, src_hbm, dst_hbm, zrow_hbm, zdeg_hbm, ones_hbm,
                  out_hbm, deg_hbm,
                  src_v, dst_v, rows_v, ones_v, acc_sh, deg_sh, sem):
    c = lax.axis_index("c")
    s = lax.axis_index("s")
    w = c * NS + s
    r0 = s * RPT

    pltpu.sync_copy(zrow_hbm, acc_sh.at[pl.ds(r0, RPT)])
    pltpu.sync_copy(zdeg_hbm, deg_sh.at[pl.ds(r0, RPT)])
    pltpu.sync_copy(ones_hbm, ones_v)
    plsc.subcore_barrier()

    base = w * EPW

    def body(i, carry):
        off = base + i * CH
        pltpu.sync_copy(src_hbm.at[pl.ds(off, CH)], src_v)
        pltpu.sync_copy(dst_hbm.at[pl.ds(off, CH)], dst_v)
        pltpu.async_copy(h_hbm.at[src_v], rows_v, sem).wait()
        pltpu.sync_copy(rows_v, acc_sh.at[dst_v], add=True)
        pltpu.sync_copy(ones_v, deg_sh.at[dst_v], add=True)
        return carry

    lax.fori_loop(0, NCHUNK, body, 0)

    plsc.subcore_barrier()
    pltpu.sync_copy(acc_sh.at[pl.ds(r0, RPT)], out_hbm.at[c, pl.ds(r0, RPT)])
    pltpu.sync_copy(deg_sh.at[pl.ds(r0, RPT)], deg_hbm.at[pl.ds(c * NP + r0, RPT)])


def _dense_body(relu, x_ref, p_ref, d_ref, ws_ref, wn_ref, b_ref, o_ref):
    deg = d_ref[:, 0:1] + d_ref[:, 1:2]
    inv = 1.0 / jnp.maximum(deg, 1.0)
    hn = (p_ref[0] + p_ref[1]) * inv
    acc = lax.dot_general(x_ref[...], ws_ref[...], (((1,), (1,)), ((), ())),
                          preferred_element_type=jnp.float32)
    acc = acc + lax.dot_general(hn, wn_ref[...], (((1,), (1,)), ((), ())),
                                preferred_element_type=jnp.float32)
    acc = acc + b_ref[0, :][None, :]
    if relu:
        acc = jnp.maximum(acc, 0.0)
    o_ref[...] = acc


def _dense(x, p, d, W_self, W_neigh, b, relu):
    bm = 1000
    grid = (N // bm,)
    return pl.pallas_call(
        functools.partial(_dense_body, relu),
        grid=grid,
        in_specs=[
            pl.BlockSpec((bm, D), lambda i: (i, 0)),
            pl.BlockSpec((NC, bm, D), lambda i: (0, i, 0)),
            pl.BlockSpec((bm, NC), lambda i: (i, 0)),
            pl.BlockSpec((D, D), lambda i: (0, 0)),
            pl.BlockSpec((D, D), lambda i: (0, 0)),
            pl.BlockSpec((1, D), lambda i: (0, 0)),
        ],
        out_specs=pl.BlockSpec((bm, D), lambda i: (i, 0)),
        out_shape=jax.ShapeDtypeStruct((N, D), jnp.float32),
    )(x, p, d, W_self, W_neigh, b.reshape(1, D))


def kernel(x, edge_index, W_neigh1, W_self1, b1, W_neigh2, W_self2, b2):
    ei = edge_index.astype(jnp.int32)
    src, dst = ei[0], ei[1]
    zrow = jnp.zeros((RPT, D), jnp.float32)
    zdeg = jnp.zeros((RPT,), jnp.float32)
    ones = jnp.ones((CH,), jnp.float32)

    p1, dflat = _sc_aggregate(x, src, dst, zrow, zdeg, ones)
    d = dflat.reshape(NC, NP).T
    h1 = _dense(x, p1, d, W_self1, W_neigh1, b1, relu=True)
    p2, _ = _sc_aggregate(h1, src, dst, zrow, zdeg, ones)
    out = _dense(h1, p2, d, W_self2, W_neigh2, b2, relu=False)
    return out


# speedup vs baseline: 12.4964x; 2.3136x over previous
import functools

import jax
import jax.numpy as jnp
from jax import lax
from jax.experimental import pallas as pl
from jax.experimental.pallas import tpu as pltpu
from jax.experimental.pallas import tpu_sc as plsc

N = 10000
D = 128
E = 320000
NC = 2
NS = 16
NW = NC * NS
EPW = E // NW
CH = 80
NCHUNK = EPW // CH
NBUF = 4
GLAG = 2
NOUT = (NCHUNK - 1) // NBUF
NP = 10240
RPT = NP // NS

_mesh = plsc.VectorSubcoreMesh(core_axis_name="c", subcore_axis_name="s")


def _sc_body(compute_deg, h_hbm, src_hbm, dst_hbm, zrow_hbm, zdeg_hbm,
             ones_hbm, out_hbm, deg_hbm,
             src_v, dst_v, rows, ones_v, acc_sh, deg_sh, sem_i, sem_g):
    c = lax.axis_index("c")
    s = lax.axis_index("s")
    w = c * NS + s
    r0 = s * RPT
    base = w * EPW

    pltpu.sync_copy(zrow_hbm, acc_sh.at[pl.ds(r0, RPT)])
    if compute_deg:
        pltpu.sync_copy(zdeg_hbm, deg_sh.at[pl.ds(r0, RPT)])
        pltpu.sync_copy(ones_hbm, ones_v)
    plsc.subcore_barrier()

    def idx_start(chunk, b):
        off = base + chunk * CH
        pltpu.async_copy(src_hbm.at[pl.ds(off, CH)], src_v[b], sem_i[b])
        pltpu.async_copy(dst_hbm.at[pl.ds(off, CH)], dst_v[b], sem_i[b])

    def idx_wait(chunk, b):
        off = base + chunk * CH
        pltpu.make_async_copy(src_hbm.at[pl.ds(off, CH)], src_v[b],
                              sem_i[b]).wait()
        pltpu.make_async_copy(dst_hbm.at[pl.ds(off, CH)], dst_v[b],
                              sem_i[b]).wait()

    def gather_start(b):
        pltpu.async_copy(h_hbm.at[src_v[b]], rows[b], sem_g[b])

    def gather_wait(b):
        pltpu.make_async_copy(h_hbm.at[src_v[b]], rows[b], sem_g[b]).wait()

    for b in range(NBUF):
        idx_start(b, b)
    for b in range(GLAG):
        idx_wait(b, b)
        gather_start(b)

    def outer(g, carry):
        for b in range(NBUF):
            chunk = g * NBUF + b
            gather_wait(b)
            pltpu.sync_copy(rows[b], acc_sh.at[dst_v[b]], add=True)
            if compute_deg:
                pltpu.sync_copy(ones_v, deg_sh.at[dst_v[b]], add=True)

            @pl.when(chunk + NBUF < NCHUNK)
            def _():
                idx_start(chunk + NBUF, b)

            @pl.when(chunk + GLAG < NCHUNK)
            def _():
                bg = (b + GLAG) % NBUF
                idx_wait(chunk + GLAG, bg)
                gather_start(bg)
        return carry

    lax.fori_loop(0, NOUT, outer, 0)

    tb = (NCHUNK - 1) % NBUF
    gather_wait(tb)
    pltpu.sync_copy(rows[tb], acc_sh.at[dst_v[tb]], add=True)
    if compute_deg:
        pltpu.sync_copy(ones_v, deg_sh.at[dst_v[tb]], add=True)

    plsc.subcore_barrier()
    pltpu.sync_copy(acc_sh.at[pl.ds(r0, RPT)], out_hbm.at[c, pl.ds(r0, RPT)])
    if compute_deg:
        pltpu.sync_copy(deg_sh.at[pl.ds(r0, RPT)],
                        deg_hbm.at[pl.ds(c * NP + r0, RPT)])


def _make_sc(compute_deg):
    return functools.partial(
        pl.kernel,
        mesh=_mesh,
        out_type=[
            jax.ShapeDtypeStruct((NC, NP, D), jnp.float32),
            jax.ShapeDtypeStruct((NC * NP,), jnp.float32),
        ],
        scratch_types=[
            [pltpu.VMEM((CH,), jnp.int32)] * NBUF,
            [pltpu.VMEM((CH,), jnp.int32)] * NBUF,
            [pltpu.VMEM((CH, D), jnp.float32)] * NBUF,
            pltpu.VMEM((CH,), jnp.float32),
            pltpu.VMEM_SHARED((NP, D), jnp.float32),
            pltpu.VMEM_SHARED((NP,), jnp.float32),
            [pltpu.SemaphoreType.DMA] * NBUF,
            [pltpu.SemaphoreType.DMA] * NBUF,
        ],
    )(functools.partial(_sc_body, compute_deg))


_sc_aggregate_deg = _make_sc(True)
_sc_aggregate_nodeg = _make_sc(False)


def _dense_body(relu, x_ref, p_ref, d_ref, ws_ref, wn_ref, b_ref, o_ref):
    deg = d_ref[:, 0:1] + d_ref[:, 1:2]
    inv = 1.0 / jnp.maximum(deg, 1.0)
    hn = (p_ref[0] + p_ref[1]) * inv
    acc = lax.dot_general(x_ref[...], ws_ref[...], (((1,), (1,)), ((), ())),
                          preferred_element_type=jnp.float32)
    acc = acc + lax.dot_general(hn, wn_ref[...], (((1,), (1,)), ((), ())),
                                preferred_element_type=jnp.float32)
    acc = acc + b_ref[0, :][None, :]
    if relu:
        acc = jnp.maximum(acc, 0.0)
    o_ref[...] = acc


def _dense(x, p, d, W_self, W_neigh, b, relu):
    bm = 1000
    grid = (N // bm,)
    return pl.pallas_call(
        functools.partial(_dense_body, relu),
        grid=grid,
        in_specs=[
            pl.BlockSpec((bm, D), lambda i: (i, 0)),
            pl.BlockSpec((NC, bm, D), lambda i: (0, i, 0)),
            pl.BlockSpec((bm, NC), lambda i: (i, 0)),
            pl.BlockSpec((D, D), lambda i: (0, 0)),
            pl.BlockSpec((D, D), lambda i: (0, 0)),
            pl.BlockSpec((1, D), lambda i: (0, 0)),
        ],
        out_specs=pl.BlockSpec((bm, D), lambda i: (i, 0)),
        out_shape=jax.ShapeDtypeStruct((N, D), jnp.float32),
    )(x, p, d, W_self, W_neigh, b.reshape(1, D))


def kernel(x, edge_index, W_neigh1, W_self1, b1, W_neigh2, W_self2, b2):
    ei = edge_index.astype(jnp.int32)
    src, dst = ei[0], ei[1]
    zrow = jnp.zeros((RPT, D), jnp.float32)
    zdeg = jnp.zeros((RPT,), jnp.float32)
    ones = jnp.ones((CH,), jnp.float32)

    p1, dflat = _sc_aggregate_deg(x, src, dst, zrow, zdeg, ones)
    d = dflat.reshape(NC, NP).T
    h1 = _dense(x, p1, d, W_self1, W_neigh1, b1, relu=True)
    p2, _ = _sc_aggregate_nodeg(h1, src, dst, zrow, zdeg, ones)
    out = _dense(h1, p2, d, W_self2, W_neigh2, b2, relu=False)
    return out


# speedup vs baseline: 14.7426x; 1.1798x over previous
import functools

import jax
import jax.numpy as jnp
from jax import lax
from jax.experimental import pallas as pl
from jax.experimental.pallas import tpu as pltpu
from jax.experimental.pallas import tpu_sc as plsc

N = 10000
D = 128
E = 320000
NC = 2
NS = 16
NW = NC * NS
EPW = E // NW
CH = 80
NCHUNK = EPW // CH
NBUF = 4
GLAG = 3
NOUT = NCHUNK // NBUF
NTAIL = NCHUNK % NBUF
NP = 10240
RPT = NP // NS

_mesh = plsc.VectorSubcoreMesh(core_axis_name="c", subcore_axis_name="s")


def _sc_body(compute_deg, h_hbm, src_hbm, dst_hbm, zrow_hbm, zdeg_hbm,
             ones_hbm, out_hbm, deg_hbm,
             src_v, dst_v, rows, ones_v, acc_sh, deg_sh, sem_i, sem_g):
    c = lax.axis_index("c")
    s = lax.axis_index("s")
    w = c * NS + s
    r0 = s * RPT
    base = w * EPW

    pltpu.sync_copy(zrow_hbm, acc_sh.at[pl.ds(r0, RPT)])
    if compute_deg:
        pltpu.sync_copy(zdeg_hbm, deg_sh.at[pl.ds(r0, RPT)])
        pltpu.sync_copy(ones_hbm, ones_v)
    plsc.subcore_barrier()

    def idx_start(chunk, b):
        off = base + chunk * CH
        pltpu.async_copy(src_hbm.at[pl.ds(off, CH)], src_v[b], sem_i[b])
        pltpu.async_copy(dst_hbm.at[pl.ds(off, CH)], dst_v[b], sem_i[b])

    def idx_wait(chunk, b):
        off = base + chunk * CH
        pltpu.make_async_copy(src_hbm.at[pl.ds(off, CH)], src_v[b],
                              sem_i[b]).wait()
        pltpu.make_async_copy(dst_hbm.at[pl.ds(off, CH)], dst_v[b],
                              sem_i[b]).wait()

    def gather_start(b):
        pltpu.async_copy(h_hbm.at[src_v[b]], rows[b], sem_g[b])

    def gather_wait(b):
        pltpu.make_async_copy(h_hbm.at[src_v[b]], rows[b], sem_g[b]).wait()

    for b in range(NBUF):
        idx_start(b, b)
    for b in range(GLAG):
        idx_wait(b, b)
        gather_start(b)

    def outer(g, carry):
        for b in range(NBUF):
            chunk = g * NBUF + b
            gather_wait(b)
            pltpu.sync_copy(rows[b], acc_sh.at[dst_v[b]], add=True)
            if compute_deg:
                pltpu.sync_copy(ones_v, deg_sh.at[dst_v[b]], add=True)

            @pl.when(chunk + NBUF < NCHUNK)
            def _():
                idx_start(chunk + NBUF, b)

            @pl.when(chunk + GLAG < NCHUNK)
            def _():
                bg = (b + GLAG) % NBUF
                idx_wait(chunk + GLAG, bg)
                gather_start(bg)
        return carry

    lax.fori_loop(0, NOUT, outer, 0)

    for t in range(NOUT * NBUF, NCHUNK):
        tb = t % NBUF
        gather_wait(tb)
        pltpu.sync_copy(rows[tb], acc_sh.at[dst_v[tb]], add=True)
        if compute_deg:
            pltpu.sync_copy(ones_v, deg_sh.at[dst_v[tb]], add=True)
        if t + GLAG < NCHUNK:
            bg = (tb + GLAG) % NBUF
            idx_wait(t + GLAG, bg)
            gather_start(bg)

    plsc.subcore_barrier()
    pltpu.sync_copy(acc_sh.at[pl.ds(r0, RPT)], out_hbm.at[c, pl.ds(r0, RPT)])
    if compute_deg:
        pltpu.sync_copy(deg_sh.at[pl.ds(r0, RPT)],
                        deg_hbm.at[pl.ds(c * NP + r0, RPT)])


def _make_sc(compute_deg):
    return functools.partial(
        pl.kernel,
        mesh=_mesh,
        out_type=[
            jax.ShapeDtypeStruct((NC, NP, D), jnp.float32),
            jax.ShapeDtypeStruct((NC * NP,), jnp.float32),
        ],
        scratch_types=[
            [pltpu.VMEM((CH,), jnp.int32)] * NBUF,
            [pltpu.VMEM((CH,), jnp.int32)] * NBUF,
            [pltpu.VMEM((CH, D), jnp.float32)] * NBUF,
            pltpu.VMEM((CH,), jnp.float32),
            pltpu.VMEM_SHARED((NP, D), jnp.float32),
            pltpu.VMEM_SHARED((NP,), jnp.float32),
            [pltpu.SemaphoreType.DMA] * NBUF,
            [pltpu.SemaphoreType.DMA] * NBUF,
        ],
    )(functools.partial(_sc_body, compute_deg))


_sc_aggregate_deg = _make_sc(True)
_sc_aggregate_nodeg = _make_sc(False)


def _dense_body(relu, x_ref, p_ref, d_ref, ws_ref, wn_ref, b_ref, o_ref):
    deg = d_ref[:, 0:1] + d_ref[:, 1:2]
    inv = 1.0 / jnp.maximum(deg, 1.0)
    hn = (p_ref[0] + p_ref[1]) * inv
    acc = lax.dot_general(x_ref[...], ws_ref[...], (((1,), (1,)), ((), ())),
                          preferred_element_type=jnp.float32)
    acc = acc + lax.dot_general(hn, wn_ref[...], (((1,), (1,)), ((), ())),
                                preferred_element_type=jnp.float32)
    acc = acc + b_ref[0, :][None, :]
    if relu:
        acc = jnp.maximum(acc, 0.0)
    o_ref[...] = acc


def _dense(x, p, d, W_self, W_neigh, b, relu):
    bm = 1000
    grid = (N // bm,)
    return pl.pallas_call(
        functools.partial(_dense_body, relu),
        grid=grid,
        in_specs=[
            pl.BlockSpec((bm, D), lambda i: (i, 0)),
            pl.BlockSpec((NC, bm, D), lambda i: (0, i, 0)),
            pl.BlockSpec((bm, NC), lambda i: (i, 0)),
            pl.BlockSpec((D, D), lambda i: (0, 0)),
            pl.BlockSpec((D, D), lambda i: (0, 0)),
            pl.BlockSpec((1, D), lambda i: (0, 0)),
        ],
        out_specs=pl.BlockSpec((bm, D), lambda i: (i, 0)),
        out_shape=jax.ShapeDtypeStruct((N, D), jnp.float32),
    )(x, p, d, W_self, W_neigh, b.reshape(1, D))


def kernel(x, edge_index, W_neigh1, W_self1, b1, W_neigh2, W_self2, b2):
    ei = edge_index.astype(jnp.int32)
    src, dst = ei[0], ei[1]
    zrow = jnp.zeros((RPT, D), jnp.float32)
    zdeg = jnp.zeros((RPT,), jnp.float32)
    ones = jnp.ones((CH,), jnp.float32)

    p1, dflat = _sc_aggregate_deg(x, src, dst, zrow, zdeg, ones)
    d = dflat.reshape(NC, NP).T
    h1 = _dense(x, p1, d, W_self1, W_neigh1, b1, relu=True)
    p2, _ = _sc_aggregate_nodeg(h1, src, dst, zrow, zdeg, ones)
    out = _dense(h1, p2, d, W_self2, W_neigh2, b2, relu=False)
    return out


# speedup vs baseline: 15.0923x; 1.0237x over previous
import functools

import jax
import jax.numpy as jnp
from jax import lax
from jax.experimental import pallas as pl
from jax.experimental.pallas import tpu as pltpu
from jax.experimental.pallas import tpu_sc as plsc

N = 10000
D = 128
E = 320000
NC = 2
NS = 16
NW = NC * NS
EPW = E // NW
CH = 80
NCHUNK = EPW // CH
NBUF = 4
NIDX = 2 * NBUF
GLAG = 3
NOUT = NCHUNK // NIDX
NP = 10240
RPT = NP // NS

_mesh = plsc.VectorSubcoreMesh(core_axis_name="c", subcore_axis_name="s")


def _sc_body(compute_deg, h_hbm, src_hbm, dst_hbm, zrow_hbm, zdeg_hbm,
             ones_hbm, out_hbm, deg_hbm,
             src_v, dst_v, rows, ones_v, acc_sh, deg_sh, sem_i, sem_g, sem_s):
    c = lax.axis_index("c")
    s = lax.axis_index("s")
    w = c * NS + s
    r0 = s * RPT
    base = w * EPW

    pltpu.sync_copy(zrow_hbm, acc_sh.at[pl.ds(r0, RPT)])
    if compute_deg:
        pltpu.sync_copy(zdeg_hbm, deg_sh.at[pl.ds(r0, RPT)])
        pltpu.sync_copy(ones_hbm, ones_v)
    plsc.subcore_barrier()

    def idx_start(chunk, bi):
        off = base + chunk * CH
        pltpu.async_copy(src_hbm.at[pl.ds(off, CH)], src_v[bi], sem_i[bi])
        pltpu.async_copy(dst_hbm.at[pl.ds(off, CH)], dst_v[bi], sem_i[bi])

    def idx_wait(chunk, bi):
        off = base + chunk * CH
        pltpu.make_async_copy(src_hbm.at[pl.ds(off, CH)], src_v[bi],
                              sem_i[bi]).wait()
        pltpu.make_async_copy(dst_hbm.at[pl.ds(off, CH)], dst_v[bi],
                              sem_i[bi]).wait()

    def gather_start(b, bi):
        pltpu.async_copy(h_hbm.at[src_v[bi]], rows[b], sem_g[b])

    def gather_wait(b, bi):
        pltpu.make_async_copy(h_hbm.at[src_v[bi]], rows[b], sem_g[b]).wait()

    def scatter_start(b, bi):
        pltpu.async_copy(rows[b], acc_sh.at[dst_v[bi]], sem_s[b], add=True)

    def scatter_wait(b, bi):
        pltpu.make_async_copy(rows[b], acc_sh.at[dst_v[bi]], sem_s[b]).wait()

    for ch0 in range(NIDX - 1):
        idx_start(ch0, ch0)
    for ch0 in range(GLAG):
        idx_wait(ch0, ch0)
        gather_start(ch0 % NBUF, ch0)

    def when(cond, fn):
        if isinstance(cond, bool):
            if cond:
                fn()
        else:
            pl.when(cond)(fn)

    def drain(chunk, k):
        b = k % NBUF
        bi = k % NIDX
        bp = (k - 1) % NBUF
        bip = (k - 1) % NIDX
        gather_wait(b, bi)
        when(chunk >= 1, lambda: scatter_wait(bp, bip))
        scatter_start(b, bi)
        if compute_deg:
            pltpu.sync_copy(ones_v, deg_sh.at[dst_v[bi]], add=True)
        when(chunk + NIDX - 1 < NCHUNK,
             lambda: idx_start(chunk + NIDX - 1, bip))

        def issue_gather():
            idx_wait(chunk + GLAG, (bi + GLAG) % NIDX)
            gather_start((b + GLAG) % NBUF, (bi + GLAG) % NIDX)

        when(chunk + GLAG < NCHUNK, issue_gather)

    def outer(g, carry):
        for k in range(NIDX):
            drain(g * NIDX + k, k)
        return carry

    lax.fori_loop(0, NOUT, outer, 0)

    for t in range(NOUT * NIDX, NCHUNK):
        drain(t, t)
    scatter_wait((NCHUNK - 1) % NBUF, (NCHUNK - 1) % NIDX)

    plsc.subcore_barrier()
    pltpu.sync_copy(acc_sh.at[pl.ds(r0, RPT)], out_hbm.at[c, pl.ds(r0, RPT)])
    if compute_deg:
        pltpu.sync_copy(deg_sh.at[pl.ds(r0, RPT)],
                        deg_hbm.at[pl.ds(c * NP + r0, RPT)])


def _make_sc(compute_deg):
    return functools.partial(
        pl.kernel,
        mesh=_mesh,
        out_type=[
            jax.ShapeDtypeStruct((NC, NP, D), jnp.float32),
            jax.ShapeDtypeStruct((NC * NP,), jnp.float32),
        ],
        scratch_types=[
            [pltpu.VMEM((CH,), jnp.int32)] * NIDX,
            [pltpu.VMEM((CH,), jnp.int32)] * NIDX,
            [pltpu.VMEM((CH, D), jnp.float32)] * NBUF,
            pltpu.VMEM((CH,), jnp.float32),
            pltpu.VMEM_SHARED((NP, D), jnp.float32),
            pltpu.VMEM_SHARED((NP,), jnp.float32),
            [pltpu.SemaphoreType.DMA] * NIDX,
            [pltpu.SemaphoreType.DMA] * NBUF,
            [pltpu.SemaphoreType.DMA] * NBUF,
        ],
    )(functools.partial(_sc_body, compute_deg))


_sc_aggregate_deg = _make_sc(True)
_sc_aggregate_nodeg = _make_sc(False)


def _dense_body(relu, x_ref, p_ref, d_ref, ws_ref, wn_ref, b_ref, o_ref):
    deg = d_ref[:, 0:1] + d_ref[:, 1:2]
    inv = 1.0 / jnp.maximum(deg, 1.0)
    hn = (p_ref[0] + p_ref[1]) * inv
    acc = lax.dot_general(x_ref[...], ws_ref[...], (((1,), (1,)), ((), ())),
                          preferred_element_type=jnp.float32)
    acc = acc + lax.dot_general(hn, wn_ref[...], (((1,), (1,)), ((), ())),
                                preferred_element_type=jnp.float32)
    acc = acc + b_ref[0, :][None, :]
    if relu:
        acc = jnp.maximum(acc, 0.0)
    o_ref[...] = acc


def _dense(x, p, d, W_self, W_neigh, b, relu):
    bm = 1000
    grid = (N // bm,)
    return pl.pallas_call(
        functools.partial(_dense_body, relu),
        grid=grid,
        in_specs=[
            pl.BlockSpec((bm, D), lambda i: (i, 0)),
            pl.BlockSpec((NC, bm, D), lambda i: (0, i, 0)),
            pl.BlockSpec((bm, NC), lambda i: (i, 0)),
            pl.BlockSpec((D, D), lambda i: (0, 0)),
            pl.BlockSpec((D, D), lambda i: (0, 0)),
            pl.BlockSpec((1, D), lambda i: (0, 0)),
        ],
        out_specs=pl.BlockSpec((bm, D), lambda i: (i, 0)),
        out_shape=jax.ShapeDtypeStruct((N, D), jnp.float32),
    )(x, p, d, W_self, W_neigh, b.reshape(1, D))


def kernel(x, edge_index, W_neigh1, W_self1, b1, W_neigh2, W_self2, b2):
    ei = edge_index.astype(jnp.int32)
    src, dst = ei[0], ei[1]
    zrow = jnp.zeros((RPT, D), jnp.float32)
    zdeg = jnp.zeros((RPT,), jnp.float32)
    ones = jnp.ones((CH,), jnp.float32)

    p1, dflat = _sc_aggregate_deg(x, src, dst, zrow, zdeg, ones)
    d = dflat.reshape(NC, NP).T
    h1 = _dense(x, p1, d, W_self1, W_neigh1, b1, relu=True)
    p2, _ = _sc_aggregate_nodeg(h1, src, dst, zrow, zdeg, ones)
    out = _dense(h1, p2, d, W_self2, W_neigh2, b2, relu=False)
    return out
